# baseline (device time: 60729 ns/iter reference)
import jax
import jax.numpy as jnp
from jax import lax
from jax.experimental import pallas as pl
from jax.experimental.pallas import tpu as pltpu

B, H, D, BS = 16, 16, 64, 16
NB = 128
PAGES = 128
HD = H * D
NK = PAGES * BS
BH = B * H
BNB = B * NB
SCALE = D ** -0.5
NEG = -1e30


def _iota2(shape, dim):
    return lax.broadcasted_iota(jnp.int32, shape, dim)


def _body(q_ref, k_ref, v_ref, bt_ref, lens_ref, out_ref,
          o_send, o_recv, m_send, m_recv, l_send, l_recv,
          send_sems, recv_sems):
    my_x = lax.axis_index("x")
    my_y = lax.axis_index("y")
    my_z = lax.axis_index("z")
    nbr = (my_x, 1 - my_y, my_z)

    barrier = pltpu.get_barrier_semaphore()
    pl.semaphore_signal(barrier, inc=1, device_id=nbr,
                        device_id_type=pl.DeviceIdType.MESH)
    pl.semaphore_wait(barrier, 1)

    maskB_bf = (_iota2((BH, HD), 1) // D == _iota2((BH, HD), 0) % H
                ).astype(jnp.bfloat16)
    maskB_f32 = (_iota2((BH, HD), 1) // D == _iota2((BH, HD), 0) % H
                 ).astype(jnp.float32)
    erep_bf = (_iota2((BH, B), 0) // H == _iota2((BH, B), 1)
               ).astype(jnp.bfloat16)
    efold_f32 = (_iota2((B, BH), 1) // H == _iota2((B, BH), 0)
                 ).astype(jnp.float32)
    gsum_bf = (_iota2((BNB, B), 0) // NB == _iota2((BNB, B), 1)
               ).astype(jnp.bfloat16)
    erow_bf = (_iota2((NK, PAGES), 0) // BS == _iota2((NK, PAGES), 1)
               ).astype(jnp.bfloat16)
    gsum_f32 = (_iota2((BNB, B), 0) // NB == _iota2((BNB, B), 1)
                ).astype(jnp.float32)

    kb = k_ref[...].astype(jnp.bfloat16)
    vb = v_ref[...].astype(jnp.bfloat16)
    qr = q_ref[...].astype(jnp.bfloat16)

    lensf = lens_ref[...].astype(jnp.float32)
    lensflat = lax.dot_general(lensf, gsum_f32, (((1,), (1,)), ((), ())),
                               preferred_element_type=jnp.float32)
    jflat = (_iota2((1, BNB), 1) % NB).astype(jnp.float32)
    valid = jflat < lensflat
    btlocal = bt_ref[...] - my_y * PAGES
    pkp = _iota2((PAGES, BNB), 0)
    match = jnp.logical_and(pkp == btlocal, valid).astype(jnp.bfloat16)
    cnt_pages = lax.dot_general(match, gsum_bf, (((1,), (0,)), ((), ())),
                                preferred_element_type=jnp.float32)
    cnt_pb = lax.dot_general(cnt_pages.astype(jnp.bfloat16), erep_bf,
                             (((1,), (1,)), ((), ())),
                             preferred_element_type=jnp.float32)
    cntcol = lax.dot_general(erow_bf, cnt_pb.astype(jnp.bfloat16),
                             (((1,), (0,)), ((), ())),
                             preferred_element_type=jnp.float32)

    qrep = lax.dot_general(erep_bf, qr, (((1,), (0,)), ((), ())),
                           preferred_element_type=jnp.float32)
    qbT = qrep.astype(jnp.bfloat16) * maskB_bf

    s = lax.dot_general(kb, qbT, (((1,), (1,)), ((), ())),
                        preferred_element_type=jnp.float32) * SCALE

    smask = jnp.where(cntcol > 0, s, NEG)
    m = jnp.max(smask, axis=0, keepdims=True)
    m_safe = jnp.where(m < -1e29, 0.0, m)
    p = jnp.exp(s - m_safe) * cntcol
    l = jnp.sum(p, axis=0, keepdims=True)

    r = lax.dot_general(p.astype(jnp.bfloat16), vb,
                        (((0,), (0,)), ((), ())),
                        preferred_element_type=jnp.float32)
    o_send[...] = jnp.dot(efold_f32, r * maskB_f32,
                          preferred_element_type=jnp.float32)
    m_send[...] = m
    l_send[...] = l

    rdmas = []
    for i, (src, dst) in enumerate(
            [(o_send, o_recv), (m_send, m_recv), (l_send, l_recv)]):
        rdma = pltpu.make_async_remote_copy(
            src_ref=src, dst_ref=dst,
            send_sem=send_sems.at[i], recv_sem=recv_sems.at[i],
            device_id=nbr, device_id_type=pl.DeviceIdType.MESH)
        rdma.start()
        rdmas.append(rdma)
    for rdma in rdmas:
        rdma.wait()

    m_loc = m_send[...]
    m_rem = m_recv[...]
    mt = jnp.maximum(m_loc, m_rem)
    a = jnp.exp(m_loc - mt)
    c = jnp.exp(m_rem - mt)
    lt = a * l_send[...] + c * l_recv[...]
    a_e = jnp.dot(a * efold_f32, maskB_f32, preferred_element_type=jnp.float32)
    c_e = jnp.dot(c * efold_f32, maskB_f32, preferred_element_type=jnp.float32)
    l_e = jnp.dot(lt * efold_f32, maskB_f32, preferred_element_type=jnp.float32)
    out_ref[...] = (a_e * o_send[...] + c_e * o_recv[...]) / l_e


def kernel(Q, K, V, bt, lens):
    Qr = Q.reshape(B, HD)
    Kr = K.reshape(NK, HD)
    Vr = V.reshape(NK, HD)
    btr = bt.reshape(1, BNB)
    lensr = lens.reshape(1, B)

    out = pl.pallas_call(
        _body,
        out_shape=jax.ShapeDtypeStruct((B, HD), jnp.float32),
        in_specs=[
            pl.BlockSpec(memory_space=pltpu.VMEM),
            pl.BlockSpec(memory_space=pltpu.VMEM),
            pl.BlockSpec(memory_space=pltpu.VMEM),
            pl.BlockSpec(memory_space=pltpu.VMEM),
            pl.BlockSpec(memory_space=pltpu.VMEM),
        ],
        out_specs=pl.BlockSpec(memory_space=pltpu.VMEM),
        scratch_shapes=[
            pltpu.VMEM((B, HD), jnp.float32),
            pltpu.VMEM((B, HD), jnp.float32),
            pltpu.VMEM((1, BH), jnp.float32),
            pltpu.VMEM((1, BH), jnp.float32),
            pltpu.VMEM((1, BH), jnp.float32),
            pltpu.VMEM((1, BH), jnp.float32),
            pltpu.SemaphoreType.DMA((3,)),
            pltpu.SemaphoreType.DMA((3,)),
        ],
        compiler_params=pltpu.CompilerParams(collective_id=0),
    )(Qr, Kr, Vr, btr, lensr)
    return out.reshape(B, 1, H, D)


# device time: 51945 ns/iter; 1.1691x vs baseline; 1.1691x over previous
import jax
import jax.numpy as jnp
from jax import lax
from jax.experimental import pallas as pl
from jax.experimental.pallas import tpu as pltpu

B, H, D, BS = 16, 16, 64, 16
NB = 128
PAGES = 128
HD = H * D
NK = PAGES * BS
BH = B * H
BNB = B * NB
SCALE = D ** -0.5
NEG = -1e30


def _iota2(shape, dim):
    return lax.broadcasted_iota(jnp.int32, shape, dim)


def _body(q_ref, k_ref, v_ref, bt_ref, lens_ref, out_ref,
          o_send, o_recv, m_send, m_recv, l_send, l_recv,
          send_sems, recv_sems):
    my_x = lax.axis_index("x")
    my_y = lax.axis_index("y")
    my_z = lax.axis_index("z")
    nbr = (my_x, 1 - my_y, my_z)

    barrier = pltpu.get_barrier_semaphore()
    pl.semaphore_signal(barrier, inc=1, device_id=nbr,
                        device_id_type=pl.DeviceIdType.MESH)
    pl.semaphore_wait(barrier, 1)

    maskB_bf = (_iota2((BH, HD), 1) // D == _iota2((BH, HD), 0) % H
                ).astype(jnp.bfloat16)
    maskB_f32 = (_iota2((BH, HD), 1) // D == _iota2((BH, HD), 0) % H
                 ).astype(jnp.float32)
    erep_bf = (_iota2((BH, B), 0) // H == _iota2((BH, B), 1)
               ).astype(jnp.bfloat16)
    efold_f32 = (_iota2((B, BH), 1) // H == _iota2((B, BH), 0)
                 ).astype(jnp.float32)
    gsum_bf = (_iota2((BNB, B), 0) // NB == _iota2((BNB, B), 1)
               ).astype(jnp.bfloat16)
    erow_bf = (_iota2((NK, PAGES), 0) // BS == _iota2((NK, PAGES), 1)
               ).astype(jnp.bfloat16)
    gsum_f32 = (_iota2((BNB, B), 0) // NB == _iota2((BNB, B), 1)
                ).astype(jnp.float32)

    kb = k_ref[...]
    vb = v_ref[...]
    qr = q_ref[...]

    lensf = lens_ref[...].astype(jnp.float32)
    lensflat = lax.dot_general(lensf, gsum_f32, (((1,), (1,)), ((), ())),
                               preferred_element_type=jnp.float32)
    jflat = (_iota2((1, BNB), 1) % NB).astype(jnp.float32)
    valid = jflat < lensflat
    btlocal = bt_ref[...] - my_y * PAGES
    pkp = _iota2((PAGES, BNB), 0)
    match = jnp.logical_and(pkp == btlocal, valid).astype(jnp.bfloat16)
    cnt_pages = lax.dot_general(match, gsum_bf, (((1,), (0,)), ((), ())),
                                preferred_element_type=jnp.float32)
    cnt_pb = lax.dot_general(cnt_pages.astype(jnp.bfloat16), erep_bf,
                             (((1,), (1,)), ((), ())),
                             preferred_element_type=jnp.float32)
    cntcol = lax.dot_general(erow_bf, cnt_pb.astype(jnp.bfloat16),
                             (((1,), (0,)), ((), ())),
                             preferred_element_type=jnp.float32)

    qrep = lax.dot_general(erep_bf, qr, (((1,), (0,)), ((), ())),
                           preferred_element_type=jnp.float32)
    qbT = qrep.astype(jnp.bfloat16) * maskB_bf

    s = lax.dot_general(kb, qbT, (((1,), (1,)), ((), ())),
                        preferred_element_type=jnp.float32) * SCALE

    smask = jnp.where(cntcol > 0, s, NEG)
    m = jnp.max(smask, axis=0, keepdims=True)
    m_safe = jnp.where(m < -1e29, 0.0, m)
    p = jnp.exp(s - m_safe) * cntcol
    l = jnp.sum(p, axis=0, keepdims=True)

    r = lax.dot_general(p.astype(jnp.bfloat16), vb,
                        (((0,), (0,)), ((), ())),
                        preferred_element_type=jnp.float32)
    o_send[...] = jnp.dot(efold_f32, r * maskB_f32,
                          preferred_element_type=jnp.float32)
    m_send[...] = m
    l_send[...] = l

    rdmas = []
    for i, (src, dst) in enumerate(
            [(o_send, o_recv), (m_send, m_recv), (l_send, l_recv)]):
        rdma = pltpu.make_async_remote_copy(
            src_ref=src, dst_ref=dst,
            send_sem=send_sems.at[i], recv_sem=recv_sems.at[i],
            device_id=nbr, device_id_type=pl.DeviceIdType.MESH)
        rdma.start()
        rdmas.append(rdma)
    for rdma in rdmas:
        rdma.wait()

    m_loc = m_send[...]
    m_rem = m_recv[...]
    mt = jnp.maximum(m_loc, m_rem)
    a = jnp.exp(m_loc - mt)
    c = jnp.exp(m_rem - mt)
    lt = a * l_send[...] + c * l_recv[...]
    a_e = jnp.dot(a * efold_f32, maskB_f32, preferred_element_type=jnp.float32)
    c_e = jnp.dot(c * efold_f32, maskB_f32, preferred_element_type=jnp.float32)
    l_e = jnp.dot(lt * efold_f32, maskB_f32, preferred_element_type=jnp.float32)
    out_ref[...] = (a_e * o_send[...] + c_e * o_recv[...]) / l_e


def kernel(Q, K, V, bt, lens):
    Qr = Q.astype(jnp.bfloat16).reshape(B, HD)
    Kr = K.astype(jnp.bfloat16).reshape(NK, HD)
    Vr = V.astype(jnp.bfloat16).reshape(NK, HD)
    btr = bt.reshape(1, BNB)
    lensr = lens.reshape(1, B)

    out = pl.pallas_call(
        _body,
        out_shape=jax.ShapeDtypeStruct((B, HD), jnp.float32),
        in_specs=[
            pl.BlockSpec(memory_space=pltpu.VMEM),
            pl.BlockSpec(memory_space=pltpu.VMEM),
            pl.BlockSpec(memory_space=pltpu.VMEM),
            pl.BlockSpec(memory_space=pltpu.VMEM),
            pl.BlockSpec(memory_space=pltpu.VMEM),
        ],
        out_specs=pl.BlockSpec(memory_space=pltpu.VMEM),
        scratch_shapes=[
            pltpu.VMEM((B, HD), jnp.float32),
            pltpu.VMEM((B, HD), jnp.float32),
            pltpu.VMEM((1, BH), jnp.float32),
            pltpu.VMEM((1, BH), jnp.float32),
            pltpu.VMEM((1, BH), jnp.float32),
            pltpu.VMEM((1, BH), jnp.float32),
            pltpu.SemaphoreType.DMA((3,)),
            pltpu.SemaphoreType.DMA((3,)),
        ],
        compiler_params=pltpu.CompilerParams(collective_id=0),
    )(Qr, Kr, Vr, btr, lensr)
    return out.reshape(B, 1, H, D)


# device time: 43819 ns/iter; 1.3859x vs baseline; 1.1854x over previous
import jax
import jax.numpy as jnp
from jax import lax
from jax.experimental import pallas as pl
from jax.experimental.pallas import tpu as pltpu

B, H, D, BS = 16, 16, 64, 16
NB = 128
PAGES = 128
HD = H * D
NK = PAGES * BS
BH = B * H
BNB = B * NB
SCALE = D ** -0.5
NEG = -1e30


def _iota2(shape, dim):
    return lax.broadcasted_iota(jnp.int32, shape, dim)


def _body(q_ref, k_ref, v_ref, bt_ref, lens_ref, out_ref,
          o_send, o_recv, m_send, m_recv, l_send, l_recv,
          send_sems, recv_sems):
    my_x = lax.axis_index("x")
    my_y = lax.axis_index("y")
    my_z = lax.axis_index("z")
    nbr = (my_x, 1 - my_y, my_z)

    barrier = pltpu.get_barrier_semaphore()
    pl.semaphore_signal(barrier, inc=1, device_id=nbr,
                        device_id_type=pl.DeviceIdType.MESH)
    pl.semaphore_wait(barrier, 1)

    maskB_bf = (_iota2((BH, HD), 1) // D == _iota2((BH, HD), 0) % H
                ).astype(jnp.bfloat16)
    maskB_f32 = (_iota2((BH, HD), 1) // D == _iota2((BH, HD), 0) % H
                 ).astype(jnp.float32)
    erep_bf = (_iota2((BH, B), 0) // H == _iota2((BH, B), 1)
               ).astype(jnp.bfloat16)
    efold_f32 = (_iota2((B, BH), 1) // H == _iota2((B, BH), 0)
                 ).astype(jnp.float32)
    gsum_bf = (_iota2((BNB, B), 0) // NB == _iota2((BNB, B), 1)
               ).astype(jnp.bfloat16)
    erow_bf = (_iota2((NK, PAGES), 0) // BS == _iota2((NK, PAGES), 1)
               ).astype(jnp.bfloat16)
    gsum_f32 = (_iota2((BNB, B), 0) // NB == _iota2((BNB, B), 1)
                ).astype(jnp.float32)

    kb = k_ref[...].reshape(NK, HD)
    vb = v_ref[...].reshape(NK, HD)
    qr = q_ref[...]

    lensf = lens_ref[...].astype(jnp.float32)
    lensflat = lax.dot_general(lensf, gsum_f32, (((1,), (1,)), ((), ())),
                               preferred_element_type=jnp.float32)
    jflat = (_iota2((1, BNB), 1) % NB).astype(jnp.float32)
    valid = jflat < lensflat
    btlocal = bt_ref[...] - my_y * PAGES
    pkp = _iota2((PAGES, BNB), 0)
    match = jnp.logical_and(pkp == btlocal, valid).astype(jnp.bfloat16)
    cnt_pages = lax.dot_general(match, gsum_bf, (((1,), (0,)), ((), ())),
                                preferred_element_type=jnp.float32)
    cnt_pb = lax.dot_general(cnt_pages.astype(jnp.bfloat16), erep_bf,
                             (((1,), (1,)), ((), ())),
                             preferred_element_type=jnp.float32)
    cntcol = lax.dot_general(erow_bf, cnt_pb.astype(jnp.bfloat16),
                             (((1,), (0,)), ((), ())),
                             preferred_element_type=jnp.float32)

    qrep = lax.dot_general(erep_bf, qr, (((1,), (0,)), ((), ())),
                           preferred_element_type=jnp.float32)
    qbT = qrep.astype(jnp.bfloat16) * maskB_bf

    s = lax.dot_general(kb, qbT, (((1,), (1,)), ((), ())),
                        preferred_element_type=jnp.float32) * SCALE

    smask = jnp.where(cntcol > 0, s, NEG)
    m = jnp.max(smask, axis=0, keepdims=True)
    m_safe = jnp.where(m < -1e29, 0.0, m)
    p = jnp.exp(s - m_safe) * cntcol
    l = jnp.sum(p, axis=0, keepdims=True)

    r = lax.dot_general(p.astype(jnp.bfloat16), vb,
                        (((0,), (0,)), ((), ())),
                        preferred_element_type=jnp.float32)
    o_send[...] = jnp.dot(efold_f32, r * maskB_f32,
                          preferred_element_type=jnp.float32)
    m_send[...] = m
    l_send[...] = l

    rdmas = []
    for i, (src, dst) in enumerate(
            [(o_send, o_recv), (m_send, m_recv), (l_send, l_recv)]):
        rdma = pltpu.make_async_remote_copy(
            src_ref=src, dst_ref=dst,
            send_sem=send_sems.at[i], recv_sem=recv_sems.at[i],
            device_id=nbr, device_id_type=pl.DeviceIdType.MESH)
        rdma.start()
        rdmas.append(rdma)
    for rdma in rdmas:
        rdma.wait()

    m_loc = m_send[...]
    m_rem = m_recv[...]
    mt = jnp.maximum(m_loc, m_rem)
    a = jnp.exp(m_loc - mt)
    c = jnp.exp(m_rem - mt)
    lt = a * l_send[...] + c * l_recv[...]
    a_e = jnp.dot(a * efold_f32, maskB_f32, preferred_element_type=jnp.float32)
    c_e = jnp.dot(c * efold_f32, maskB_f32, preferred_element_type=jnp.float32)
    l_e = jnp.dot(lt * efold_f32, maskB_f32, preferred_element_type=jnp.float32)
    out_ref[...] = (a_e * o_send[...] + c_e * o_recv[...]) / l_e


def kernel(Q, K, V, bt, lens):
    Qr = Q.astype(jnp.bfloat16).reshape(B, HD)
    Kr = K.astype(jnp.bfloat16).reshape(NK, H, D)
    Vr = V.astype(jnp.bfloat16).reshape(NK, H, D)
    btr = bt.reshape(1, BNB)
    lensr = lens.reshape(1, B)

    out = pl.pallas_call(
        _body,
        out_shape=jax.ShapeDtypeStruct((B, HD), jnp.float32),
        in_specs=[
            pl.BlockSpec(memory_space=pltpu.VMEM),
            pl.BlockSpec(memory_space=pltpu.VMEM),
            pl.BlockSpec(memory_space=pltpu.VMEM),
            pl.BlockSpec(memory_space=pltpu.VMEM),
            pl.BlockSpec(memory_space=pltpu.VMEM),
        ],
        out_specs=pl.BlockSpec(memory_space=pltpu.VMEM),
        scratch_shapes=[
            pltpu.VMEM((B, HD), jnp.float32),
            pltpu.VMEM((B, HD), jnp.float32),
            pltpu.VMEM((1, BH), jnp.float32),
            pltpu.VMEM((1, BH), jnp.float32),
            pltpu.VMEM((1, BH), jnp.float32),
            pltpu.VMEM((1, BH), jnp.float32),
            pltpu.SemaphoreType.DMA((3,)),
            pltpu.SemaphoreType.DMA((3,)),
        ],
        compiler_params=pltpu.CompilerParams(collective_id=0),
    )(Qr, Kr, Vr, btr, lensr)
    return out.reshape(B, 1, H, D)
